# baseline (speedup 1.0000x reference)
import jax
import jax.numpy as jnp
from jax.experimental import pallas as pl
from jax.experimental.pallas import tpu as pltpu

_T = 2048
_D = 1024
_E = 8
_K = 2
_F = 256
_BT = 512
_NE = _E + 1
_FC = _NE * _F


def _moe_block_kernel(x_ref, gate_wt_ref, corr_ref, wg_ref, wu_ref, wd_ref,
                      sg_ref, su_ref, sd_ref, out_ref,
                      wg_scr, wu_scr, wd_scr):
    i = pl.program_id(0)
    bf = jnp.bfloat16

    @pl.when(i == 0)
    def _build_panels():
        for e in range(_E):
            wg_scr[:, e * _F:(e + 1) * _F] = wg_ref[e].astype(bf)
            wu_scr[:, e * _F:(e + 1) * _F] = wu_ref[e].astype(bf)
            wd_scr[e * _F:(e + 1) * _F, :] = wd_ref[e].astype(bf)
        wg_scr[:, _E * _F:] = sg_ref[...].astype(bf)
        wu_scr[:, _E * _F:] = su_ref[...].astype(bf)
        wd_scr[_E * _F:, :] = sd_ref[...].astype(bf)

    x = x_ref[...].astype(bf)

    logits = jnp.dot(x, gate_wt_ref[...].astype(bf),
                     preferred_element_type=jnp.float32)
    hg = jnp.dot(x, wg_scr[...], preferred_element_type=jnp.float32)
    hu = jnp.dot(x, wu_scr[...], preferred_element_type=jnp.float32)

    scores = jax.nn.softmax(logits, axis=-1)
    biased = scores + corr_ref[...]

    eidx = jax.lax.broadcasted_iota(jnp.int32, (_BT, _E), 1)
    m1 = jnp.max(biased, axis=-1, keepdims=True)
    i1 = jnp.min(jnp.where(biased == m1, eidx, _E), axis=-1, keepdims=True)
    b2 = jnp.where(eidx == i1, -jnp.inf, biased)
    m2 = jnp.max(b2, axis=-1, keepdims=True)
    i2 = jnp.min(jnp.where(b2 == m2, eidx, _E), axis=-1, keepdims=True)
    w1 = jnp.sum(jnp.where(eidx == i1, scores, 0.0), axis=-1, keepdims=True)
    w2 = jnp.sum(jnp.where(eidx == i2, scores, 0.0), axis=-1, keepdims=True)
    denom = w1 + w2
    route = (jnp.where(eidx == i1, w1, 0.0)
             + jnp.where(eidx == i2, w2, 0.0)) / denom
    route = route.astype(bf).astype(jnp.float32)

    act = jax.nn.silu(hg) * hu
    parts = [act[:, e * _F:(e + 1) * _F] * route[:, e:e + 1]
             for e in range(_E)]
    parts.append(act[:, _E * _F:])
    inter = jnp.concatenate(parts, axis=1).astype(bf)
    out_ref[...] = jnp.dot(inter, wd_scr[...],
                           preferred_element_type=jnp.float32)


def kernel(hidden_states, visual_token_mask, gate_w, corr_bias, w_gate, w_up,
           w_down, sh_gate, sh_up, sh_down):
    x = hidden_states.reshape(-1, _D)
    gate_wt = gate_w.T
    corr = corr_bias.reshape(1, _E)

    grid = (_T // _BT,)
    out = pl.pallas_call(
        _moe_block_kernel,
        grid=grid,
        in_specs=[
            pl.BlockSpec((_BT, _D), lambda i: (i, 0)),
            pl.BlockSpec((_D, _E), lambda i: (0, 0)),
            pl.BlockSpec((1, _E), lambda i: (0, 0)),
            pl.BlockSpec((_E, _D, _F), lambda i: (0, 0, 0)),
            pl.BlockSpec((_E, _D, _F), lambda i: (0, 0, 0)),
            pl.BlockSpec((_E, _F, _D), lambda i: (0, 0, 0)),
            pl.BlockSpec((_D, _F), lambda i: (0, 0)),
            pl.BlockSpec((_D, _F), lambda i: (0, 0)),
            pl.BlockSpec((_F, _D), lambda i: (0, 0)),
        ],
        out_specs=pl.BlockSpec((_BT, _D), lambda i: (i, 0)),
        out_shape=jax.ShapeDtypeStruct((_T, _D), jnp.float32),
        scratch_shapes=[
            pltpu.VMEM((_D, _FC), jnp.bfloat16),
            pltpu.VMEM((_D, _FC), jnp.bfloat16),
            pltpu.VMEM((_FC, _D), jnp.bfloat16),
        ],
        compiler_params=pltpu.CompilerParams(
            dimension_semantics=("arbitrary",)),
    )(x, gate_wt, corr, w_gate, w_up, w_down, sh_gate, sh_up, sh_down)
    return out.reshape(hidden_states.shape)
